# Initial kernel scaffold; baseline (speedup 1.0000x reference)
import jax
import jax.numpy as jnp
from jax.experimental import pallas as pl


def kernel(rois, feature, stride):
    raise NotImplementedError("write your pallas kernel here")


# speedup vs baseline: 1.4850x; 1.4850x over previous
import jax
import jax.numpy as jnp
from jax.experimental import pallas as pl
from jax.experimental.pallas import tpu as pltpu

_P = 7
_K = 16
_KW = 24


def _roi_kernel(meta_ref, feat_hbm, out_ref, crop, wp, res, sems):
    i = pl.program_id(0)
    n = pl.num_programs(0)
    slot = jax.lax.rem(i, 2)
    nslot = jax.lax.rem(i + 1, 2)

    def copy_for(roi, s):
        rb = meta_ref[roi, 0]
        xs = pl.multiple_of(meta_ref[roi, 1], 8)
        return pltpu.make_async_copy(
            feat_hbm.at[pl.ds(rb, _K), pl.ds(xs, _KW), :],
            crop.at[s], sems.at[s])

    @pl.when(i == 0)
    def _():
        copy_for(0, 0).start()

    @pl.when(i + 1 < n)
    def _():
        copy_for(i + 1, nslot).start()

    copy_for(i, slot).wait()

    c = crop.shape[-1]
    neg = jnp.full((_K, 1, c), -jnp.inf, jnp.float32)

    for q in range(_P):
        cs = meta_ref[i, 16 + q]
        ce = meta_ref[i, 23 + q]
        span = ce - cs
        c0 = jnp.minimum(cs, _KW - 1)
        c1 = jnp.minimum(cs + 1, _KW - 1)
        c2 = jnp.minimum(cs + 2, _KW - 1)
        a0 = crop[slot, :, pl.ds(c0, 1), :]
        a1 = crop[slot, :, pl.ds(c1, 1), :]
        a2 = crop[slot, :, pl.ds(c2, 1), :]
        m = a0
        m = jnp.where(span >= 2, jnp.maximum(m, a1), m)
        m = jnp.where(span >= 3, jnp.maximum(m, a2), m)
        m = jnp.where(span >= 1, m, neg)
        wp[:, pl.ds(q, 1), :] = m

    for p in range(_P):
        rs = meta_ref[i, 2 + p]
        re = meta_ref[i, 9 + p]
        span = re - rs
        r0 = jnp.minimum(rs, _K - 1)
        r1 = jnp.minimum(rs + 1, _K - 1)
        r2 = jnp.minimum(rs + 2, _K - 1)
        b0 = wp[pl.ds(r0, 1), :, :]
        b1 = wp[pl.ds(r1, 1), :, :]
        b2 = wp[pl.ds(r2, 1), :, :]
        m = b0
        m = jnp.where(span >= 2, jnp.maximum(m, b1), m)
        m = jnp.where(span >= 3, jnp.maximum(m, b2), m)
        m = jnp.where(span >= 1, m, jnp.full((1, _P, c), -jnp.inf, jnp.float32))
        res[pl.ds(_P * p, _P), :] = m.reshape(_P, c)

    v = res[:, :]
    v = jnp.where(v == -jnp.inf, jnp.float32(0.0), v)
    out_ref[0] = v.T


def kernel(rois, feature, stride):
    B, C, H, W = feature.shape
    N = rois.shape[0]
    f32 = jnp.float32
    scale = (1.0 / stride)

    r = rois[:, :5].astype(f32)
    b = r[:, 0].astype(jnp.int32)
    x1 = jnp.round(r[:, 1] * scale).astype(jnp.int32)
    y1 = jnp.round(r[:, 2] * scale).astype(jnp.int32)
    x2 = jnp.round(r[:, 3] * scale).astype(jnp.int32)
    y2 = jnp.round(r[:, 4] * scale).astype(jnp.int32)
    roi_w = jnp.maximum(x2 - x1 + 1, 1)
    roi_h = jnp.maximum(y2 - y1 + 1, 1)
    bin_h = roi_h.astype(f32) / _P
    bin_w = roi_w.astype(f32) / _P
    ph = jnp.arange(_P, dtype=f32)
    hs = jnp.clip(jnp.floor(ph[None, :] * bin_h[:, None]).astype(jnp.int32)
                  + y1[:, None], 0, H)
    he = jnp.clip(jnp.ceil((ph[None, :] + 1.0) * bin_h[:, None]).astype(jnp.int32)
                  + y1[:, None], 0, H)
    ws = jnp.clip(jnp.floor(ph[None, :] * bin_w[:, None]).astype(jnp.int32)
                  + x1[:, None], 0, W)
    we = jnp.clip(jnp.ceil((ph[None, :] + 1.0) * bin_w[:, None]).astype(jnp.int32)
                  + x1[:, None], 0, W)
    ys = jnp.minimum(y1, H - _K)
    xs = jnp.minimum((jnp.minimum(x1, W - _K) // 8) * 8, W - _KW)
    rowbase = b * H + ys
    meta = jnp.concatenate(
        [rowbase[:, None], xs[:, None],
         hs - ys[:, None], he - ys[:, None],
         ws - xs[:, None], we - xs[:, None],
         jnp.zeros((N, 2), jnp.int32)], axis=1)

    feat_t = jnp.transpose(feature, (0, 2, 3, 1)).reshape(B * H, W, C)

    grid_spec = pltpu.PrefetchScalarGridSpec(
        num_scalar_prefetch=1,
        grid=(N,),
        in_specs=[pl.BlockSpec(memory_space=pl.ANY)],
        out_specs=pl.BlockSpec((1, C, _P * _P), lambda i, meta: (i, 0, 0)),
        scratch_shapes=[
            pltpu.VMEM((2, _K, _KW, C), f32),
            pltpu.VMEM((_K, _P, C), f32),
            pltpu.VMEM((_P * _P, C), f32),
            pltpu.SemaphoreType.DMA((2,)),
        ],
    )
    out = pl.pallas_call(
        _roi_kernel,
        grid_spec=grid_spec,
        out_shape=jax.ShapeDtypeStruct((N, C, _P * _P), f32),
    )(meta, feat_t)
    return out.reshape(N, C, _P, _P)


# speedup vs baseline: 2.1148x; 1.4241x over previous
import jax
import jax.numpy as jnp
from jax.experimental import pallas as pl
from jax.experimental.pallas import tpu as pltpu

_P = 7
_K = 16
_KW = 24
_G = 2


def _roi_kernel(meta_ref, feat_hbm, out_ref, crop, wp, res, sems):
    i = pl.program_id(0)
    n = pl.num_programs(0)
    slot = jax.lax.rem(i, 2)
    nslot = jax.lax.rem(i + 1, 2)

    def copy_for(step, s, j):
        roi = _G * step + j
        rb = meta_ref[roi, 0]
        xs = pl.multiple_of(meta_ref[roi, 1], 8)
        return pltpu.make_async_copy(
            feat_hbm.at[pl.ds(rb, _K), pl.ds(xs, _KW), :],
            crop.at[s, j], sems.at[s, j])

    @pl.when(i == 0)
    def _():
        for j in range(_G):
            copy_for(0, 0, j).start()

    @pl.when(i + 1 < n)
    def _():
        for j in range(_G):
            copy_for(i + 1, nslot, j).start()

    for j in range(_G):
        copy_for(i, slot, j).wait()

    c = crop.shape[-1]
    neg1 = jnp.full((_K, 1, c), -jnp.inf, jnp.float32)
    neg2 = jnp.full((1, _P, c), -jnp.inf, jnp.float32)

    for j in range(_G):
        roi = _G * i + j
        for q in range(_P):
            cs = meta_ref[roi, 16 + q]
            ce = meta_ref[roi, 23 + q]
            span = ce - cs
            c0 = jnp.minimum(cs, _KW - 1)
            c1 = jnp.minimum(cs + 1, _KW - 1)
            c2 = jnp.minimum(cs + 2, _KW - 1)
            a0 = crop[slot, j, :, pl.ds(c0, 1), :]
            a1 = crop[slot, j, :, pl.ds(c1, 1), :]
            a2 = crop[slot, j, :, pl.ds(c2, 1), :]
            m = a0
            m = jnp.where(span >= 2, jnp.maximum(m, a1), m)
            m = jnp.where(span >= 3, jnp.maximum(m, a2), m)
            m = jnp.where(span >= 1, m, neg1)
            wp[j, :, pl.ds(q, 1), :] = m

        for p in range(_P):
            rs = meta_ref[roi, 2 + p]
            re = meta_ref[roi, 9 + p]
            span = re - rs
            r0 = jnp.minimum(rs, _K - 1)
            r1 = jnp.minimum(rs + 1, _K - 1)
            r2 = jnp.minimum(rs + 2, _K - 1)
            b0 = wp[j, pl.ds(r0, 1), :, :]
            b1 = wp[j, pl.ds(r1, 1), :, :]
            b2 = wp[j, pl.ds(r2, 1), :, :]
            m = b0
            m = jnp.where(span >= 2, jnp.maximum(m, b1), m)
            m = jnp.where(span >= 3, jnp.maximum(m, b2), m)
            m = jnp.where(span >= 1, m, neg2)
            res[j, pl.ds(_P * p, _P), :] = m.reshape(_P, c)

    for j in range(_G):
        v = res[j, :, :]
        v = jnp.where(v == -jnp.inf, jnp.float32(0.0), v)
        out_ref[j] = v.T


def kernel(rois, feature, stride):
    B, C, H, W = feature.shape
    N = rois.shape[0]
    f32 = jnp.float32
    scale = (1.0 / stride)

    r = rois[:, :5].astype(f32)
    b = r[:, 0].astype(jnp.int32)
    x1 = jnp.round(r[:, 1] * scale).astype(jnp.int32)
    y1 = jnp.round(r[:, 2] * scale).astype(jnp.int32)
    x2 = jnp.round(r[:, 3] * scale).astype(jnp.int32)
    y2 = jnp.round(r[:, 4] * scale).astype(jnp.int32)
    roi_w = jnp.maximum(x2 - x1 + 1, 1)
    roi_h = jnp.maximum(y2 - y1 + 1, 1)
    bin_h = roi_h.astype(f32) / _P
    bin_w = roi_w.astype(f32) / _P
    ph = jnp.arange(_P, dtype=f32)
    hs = jnp.clip(jnp.floor(ph[None, :] * bin_h[:, None]).astype(jnp.int32)
                  + y1[:, None], 0, H)
    he = jnp.clip(jnp.ceil((ph[None, :] + 1.0) * bin_h[:, None]).astype(jnp.int32)
                  + y1[:, None], 0, H)
    ws = jnp.clip(jnp.floor(ph[None, :] * bin_w[:, None]).astype(jnp.int32)
                  + x1[:, None], 0, W)
    we = jnp.clip(jnp.ceil((ph[None, :] + 1.0) * bin_w[:, None]).astype(jnp.int32)
                  + x1[:, None], 0, W)
    ys = jnp.minimum(y1, H - _K)
    xs = jnp.minimum((jnp.minimum(x1, W - _K) // 8) * 8, W - _KW)
    rowbase = b * H + ys
    meta = jnp.concatenate(
        [rowbase[:, None], xs[:, None],
         hs - ys[:, None], he - ys[:, None],
         ws - xs[:, None], we - xs[:, None],
         jnp.zeros((N, 2), jnp.int32)], axis=1)

    feat_t = jnp.transpose(feature, (0, 2, 3, 1)).reshape(B * H, W, C)

    grid_spec = pltpu.PrefetchScalarGridSpec(
        num_scalar_prefetch=1,
        grid=(N // _G,),
        in_specs=[pl.BlockSpec(memory_space=pl.ANY)],
        out_specs=pl.BlockSpec((_G, C, _P * _P), lambda i, meta: (i, 0, 0)),
        scratch_shapes=[
            pltpu.VMEM((2, _G, _K, _KW, C), f32),
            pltpu.VMEM((_G, _K, _P, C), f32),
            pltpu.VMEM((_G, _P * _P, C), f32),
            pltpu.SemaphoreType.DMA((2, _G)),
        ],
    )
    out = pl.pallas_call(
        _roi_kernel,
        grid_spec=grid_spec,
        out_shape=jax.ShapeDtypeStruct((N, C, _P * _P), f32),
    )(meta, feat_t)
    return out.reshape(N, C, _P, _P)


# speedup vs baseline: 2.1157x; 1.0004x over previous
import jax
import jax.numpy as jnp
from jax.experimental import pallas as pl
from jax.experimental.pallas import tpu as pltpu

_P = 7
_K = 16
_KW = 24
_G = 2


def _roi_kernel(meta_ref, feat_hbm, out_ref, crop, wp, res, sems):
    i = pl.program_id(0)
    n = pl.num_programs(0)
    slot = jax.lax.rem(i, 2)
    nslot = jax.lax.rem(i + 1, 2)

    def copy_for(step, s, j):
        roi = _G * step + j
        rb = meta_ref[roi, 0]
        xs = pl.multiple_of(meta_ref[roi, 1], 8)
        return pltpu.make_async_copy(
            feat_hbm.at[pl.ds(rb, _K), pl.ds(xs, _KW), :],
            crop.at[_G * s + j], sems.at[_G * s + j])

    @pl.when(i == 0)
    def _():
        for j in range(_G):
            copy_for(0, 0, j).start()

    @pl.when(i + 1 < n)
    def _():
        for j in range(_G):
            copy_for(i + 1, nslot, j).start()

    for j in range(_G):
        copy_for(i, slot, j).wait()

    c = crop.shape[-1]
    neg1 = jnp.full((_K, 1, c), -jnp.inf, jnp.float32)
    neg2 = jnp.full((1, _P, c), -jnp.inf, jnp.float32)

    for j in range(_G):
        roi = _G * i + j
        for q in range(_P):
            cs = meta_ref[roi, 16 + q]
            ce = meta_ref[roi, 23 + q]
            span = ce - cs
            c0 = jnp.minimum(cs, _KW - 1)
            c1 = jnp.minimum(cs + 1, _KW - 1)
            c2 = jnp.minimum(cs + 2, _KW - 1)
            bi = _G * slot + j
            a0 = crop[bi, :, pl.ds(c0, 1), :]
            a1 = crop[bi, :, pl.ds(c1, 1), :]
            a2 = crop[bi, :, pl.ds(c2, 1), :]
            m = a0
            m = jnp.where(span >= 2, jnp.maximum(m, a1), m)
            m = jnp.where(span >= 3, jnp.maximum(m, a2), m)
            m = jnp.where(span >= 1, m, neg1)
            wp[j, :, pl.ds(q, 1), :] = m

        for p in range(_P):
            rs = meta_ref[roi, 2 + p]
            re = meta_ref[roi, 9 + p]
            span = re - rs
            r0 = jnp.minimum(rs, _K - 1)
            r1 = jnp.minimum(rs + 1, _K - 1)
            r2 = jnp.minimum(rs + 2, _K - 1)
            b0 = wp[j, pl.ds(r0, 1), :, :]
            b1 = wp[j, pl.ds(r1, 1), :, :]
            b2 = wp[j, pl.ds(r2, 1), :, :]
            m = b0
            m = jnp.where(span >= 2, jnp.maximum(m, b1), m)
            m = jnp.where(span >= 3, jnp.maximum(m, b2), m)
            m = jnp.where(span >= 1, m, neg2)
            res[j, pl.ds(_P * p, _P), :] = m.reshape(_P, c)

    for j in range(_G):
        v = res[j, :, :]
        v = jnp.where(v == -jnp.inf, jnp.float32(0.0), v)
        out_ref[j] = v.T


def kernel(rois, feature, stride):
    B, C, H, W = feature.shape
    N = rois.shape[0]
    f32 = jnp.float32
    scale = (1.0 / stride)

    r = rois[:, :5].astype(f32)
    b = r[:, 0].astype(jnp.int32)
    x1 = jnp.round(r[:, 1] * scale).astype(jnp.int32)
    y1 = jnp.round(r[:, 2] * scale).astype(jnp.int32)
    x2 = jnp.round(r[:, 3] * scale).astype(jnp.int32)
    y2 = jnp.round(r[:, 4] * scale).astype(jnp.int32)
    roi_w = jnp.maximum(x2 - x1 + 1, 1)
    roi_h = jnp.maximum(y2 - y1 + 1, 1)
    bin_h = roi_h.astype(f32) / _P
    bin_w = roi_w.astype(f32) / _P
    ph = jnp.arange(_P, dtype=f32)
    hs = jnp.clip(jnp.floor(ph[None, :] * bin_h[:, None]).astype(jnp.int32)
                  + y1[:, None], 0, H)
    he = jnp.clip(jnp.ceil((ph[None, :] + 1.0) * bin_h[:, None]).astype(jnp.int32)
                  + y1[:, None], 0, H)
    ws = jnp.clip(jnp.floor(ph[None, :] * bin_w[:, None]).astype(jnp.int32)
                  + x1[:, None], 0, W)
    we = jnp.clip(jnp.ceil((ph[None, :] + 1.0) * bin_w[:, None]).astype(jnp.int32)
                  + x1[:, None], 0, W)
    ys = jnp.minimum(y1, H - _K)
    xs = jnp.minimum((jnp.minimum(x1, W - _K) // 8) * 8, W - _KW)
    rowbase = b * H + ys
    meta = jnp.concatenate(
        [rowbase[:, None], xs[:, None],
         hs - ys[:, None], he - ys[:, None],
         ws - xs[:, None], we - xs[:, None],
         jnp.zeros((N, 2), jnp.int32)], axis=1)

    feat_t = jnp.transpose(feature, (0, 2, 3, 1)).reshape(B * H, W, C)

    grid_spec = pltpu.PrefetchScalarGridSpec(
        num_scalar_prefetch=1,
        grid=(N // _G,),
        in_specs=[pl.BlockSpec(memory_space=pl.ANY)],
        out_specs=pl.BlockSpec((_G, C, _P * _P), lambda i, meta: (i, 0, 0)),
        scratch_shapes=[
            pltpu.VMEM((2 * _G, _K, _KW, C), f32),
            pltpu.VMEM((_G, _K, _P, C), f32),
            pltpu.VMEM((_G, _P * _P, C), f32),
            pltpu.SemaphoreType.DMA((2 * _G,)),
        ],
    )
    out = pl.pallas_call(
        _roi_kernel,
        grid_spec=grid_spec,
        out_shape=jax.ShapeDtypeStruct((N, C, _P * _P), f32),
    )(meta, feat_t)
    return out.reshape(N, C, _P, _P)


# speedup vs baseline: 2.7804x; 1.3141x over previous
import jax
import jax.numpy as jnp
from jax.experimental import pallas as pl
from jax.experimental.pallas import tpu as pltpu

_P = 7
_K = 16
_KW = 24
_G = 4


def _roi_kernel(meta_ref, feat_hbm, out_ref, crop, wp, res, sems):
    i = pl.program_id(0)
    n = pl.num_programs(0)
    slot = jax.lax.rem(i, 2)
    nslot = jax.lax.rem(i + 1, 2)

    def copy_for(step, s, j):
        roi = _G * step + j
        rb = meta_ref[roi, 0]
        xs = pl.multiple_of(meta_ref[roi, 1], 8)
        return pltpu.make_async_copy(
            feat_hbm.at[pl.ds(rb, _K), pl.ds(xs, _KW), :],
            crop.at[_G * s + j], sems.at[_G * s + j])

    @pl.when(i == 0)
    def _():
        for j in range(_G):
            copy_for(0, 0, j).start()

    @pl.when(i + 1 < n)
    def _():
        for j in range(_G):
            copy_for(i + 1, nslot, j).start()

    for j in range(_G):
        copy_for(i, slot, j).wait()

    c = crop.shape[-1]

    for j in range(_G):
        roi = _G * i + j
        bi = _G * slot + j
        neg1 = jnp.full((_K, 1, c), -jnp.inf, jnp.float32)
        for q in range(_P):
            cs = meta_ref[roi, 16 + q]
            ce = meta_ref[roi, 23 + q]
            span = ce - cs
            c0 = jnp.minimum(cs, _KW - 1)
            c1 = jnp.minimum(cs + 1, _KW - 1)
            c2 = jnp.minimum(cs + 2, _KW - 1)
            a0 = crop[bi, :, pl.ds(c0, 1), :]
            a1 = crop[bi, :, pl.ds(c1, 1), :]
            a2 = crop[bi, :, pl.ds(c2, 1), :]
            m = a0
            m = jnp.where(span >= 2, jnp.maximum(m, a1), m)
            m = jnp.where(span >= 3, jnp.maximum(m, a2), m)
            m = jnp.where(span >= 1, m, neg1)
            wp[j, :, pl.ds(q, 1), :] = m

        iota_h = jax.lax.broadcasted_iota(jnp.int32, (3, 1, 1), 0)
        for p in range(_P):
            rs = meta_ref[roi, 2 + p]
            re = meta_ref[roi, 9 + p]
            r0 = jnp.minimum(rs, _K - 3)
            lo = rs - r0
            hi = re - r0
            b = wp[j, pl.ds(r0, 3), :, :]
            b = jnp.where((iota_h >= lo) & (iota_h < hi), b, -jnp.inf)
            m = jnp.max(b, axis=0)
            res[j, pl.ds(_P * p, _P), :] = m

    for j in range(_G):
        v = res[j, :, :]
        v = jnp.where(v == -jnp.inf, jnp.float32(0.0), v)
        out_ref[j] = v.T


def kernel(rois, feature, stride):
    B, C, H, W = feature.shape
    N = rois.shape[0]
    f32 = jnp.float32
    scale = (1.0 / stride)

    r = rois[:, :5].astype(f32)
    b = r[:, 0].astype(jnp.int32)
    x1 = jnp.round(r[:, 1] * scale).astype(jnp.int32)
    y1 = jnp.round(r[:, 2] * scale).astype(jnp.int32)
    x2 = jnp.round(r[:, 3] * scale).astype(jnp.int32)
    y2 = jnp.round(r[:, 4] * scale).astype(jnp.int32)
    roi_w = jnp.maximum(x2 - x1 + 1, 1)
    roi_h = jnp.maximum(y2 - y1 + 1, 1)
    bin_h = roi_h.astype(f32) / _P
    bin_w = roi_w.astype(f32) / _P
    ph = jnp.arange(_P, dtype=f32)
    hs = jnp.clip(jnp.floor(ph[None, :] * bin_h[:, None]).astype(jnp.int32)
                  + y1[:, None], 0, H)
    he = jnp.clip(jnp.ceil((ph[None, :] + 1.0) * bin_h[:, None]).astype(jnp.int32)
                  + y1[:, None], 0, H)
    ws = jnp.clip(jnp.floor(ph[None, :] * bin_w[:, None]).astype(jnp.int32)
                  + x1[:, None], 0, W)
    we = jnp.clip(jnp.ceil((ph[None, :] + 1.0) * bin_w[:, None]).astype(jnp.int32)
                  + x1[:, None], 0, W)
    ys = jnp.minimum(y1, H - _K)
    xs = jnp.minimum((jnp.minimum(x1, W - _K) // 8) * 8, W - _KW)
    rowbase = b * H + ys
    meta = jnp.concatenate(
        [rowbase[:, None], xs[:, None],
         hs - ys[:, None], he - ys[:, None],
         ws - xs[:, None], we - xs[:, None],
         jnp.zeros((N, 2), jnp.int32)], axis=1)

    feat_t = jnp.transpose(feature, (0, 2, 3, 1)).reshape(B * H, W, C)

    grid_spec = pltpu.PrefetchScalarGridSpec(
        num_scalar_prefetch=1,
        grid=(N // _G,),
        in_specs=[pl.BlockSpec(memory_space=pl.ANY)],
        out_specs=pl.BlockSpec((_G, C, _P * _P), lambda i, meta: (i, 0, 0)),
        scratch_shapes=[
            pltpu.VMEM((2 * _G, _K, _KW, C), f32),
            pltpu.VMEM((_G, _K, _P, C), f32),
            pltpu.VMEM((_G, _P * _P, C), f32),
            pltpu.SemaphoreType.DMA((2 * _G,)),
        ],
    )
    out = pl.pallas_call(
        _roi_kernel,
        grid_spec=grid_spec,
        out_shape=jax.ShapeDtypeStruct((N, C, _P * _P), f32),
    )(meta, feat_t)
    return out.reshape(N, C, _P, _P)


# speedup vs baseline: 3.2740x; 1.1776x over previous
import jax
import jax.numpy as jnp
from jax.experimental import pallas as pl
from jax.experimental.pallas import tpu as pltpu

_P = 7
_K = 16
_KW = 24
_G = 8


def _roi_kernel(meta_ref, feat_hbm, out_ref, crop, wp, res, sems):
    i = pl.program_id(0)
    n = pl.num_programs(0)
    slot = jax.lax.rem(i, 2)
    nslot = jax.lax.rem(i + 1, 2)

    def copy_for(step, s, j):
        roi = _G * step + j
        rb = meta_ref[roi, 0]
        xs = pl.multiple_of(meta_ref[roi, 1], 8)
        return pltpu.make_async_copy(
            feat_hbm.at[pl.ds(rb, _K), pl.ds(xs, _KW), :],
            crop.at[_G * s + j], sems.at[_G * s + j])

    @pl.when(i == 0)
    def _():
        for j in range(_G):
            copy_for(0, 0, j).start()

    @pl.when(i + 1 < n)
    def _():
        for j in range(_G):
            copy_for(i + 1, nslot, j).start()

    for j in range(_G):
        copy_for(i, slot, j).wait()

    c = crop.shape[-1]

    for j in range(_G):
        roi = _G * i + j
        bi = _G * slot + j
        neg1 = jnp.full((_K, 1, c), -jnp.inf, jnp.float32)
        for q in range(_P):
            c0 = meta_ref[roi, 16 + 4 * q]
            c1 = meta_ref[roi, 17 + 4 * q]
            c2 = meta_ref[roi, 18 + 4 * q]
            span = meta_ref[roi, 19 + 4 * q]
            a0 = crop[bi, :, pl.ds(c0, 1), :]
            a1 = crop[bi, :, pl.ds(c1, 1), :]
            a2 = crop[bi, :, pl.ds(c2, 1), :]
            m = a0
            m = jnp.where(span >= 2, jnp.maximum(m, a1), m)
            m = jnp.where(span >= 3, jnp.maximum(m, a2), m)
            m = jnp.where(span >= 1, m, neg1)
            wp[j, :, pl.ds(q, 1), :] = m

        iota_h = jax.lax.broadcasted_iota(jnp.int32, (3, 1, 1), 0)
        for p in range(_P):
            r0 = meta_ref[roi, 44 + 3 * p]
            lo = meta_ref[roi, 45 + 3 * p]
            hi = meta_ref[roi, 46 + 3 * p]
            b = wp[j, pl.ds(r0, 3), :, :]
            b = jnp.where((iota_h >= lo) & (iota_h < hi), b, -jnp.inf)
            m = jnp.max(b, axis=0)
            res[j, pl.ds(_P * p, _P), :] = m

    for j in range(_G):
        v = res[j, :, :]
        v = jnp.where(v == -jnp.inf, jnp.float32(0.0), v)
        out_ref[j] = v.T


def kernel(rois, feature, stride):
    B, C, H, W = feature.shape
    N = rois.shape[0]
    f32 = jnp.float32
    scale = (1.0 / stride)

    r = rois[:, :5].astype(f32)
    b = r[:, 0].astype(jnp.int32)
    x1 = jnp.round(r[:, 1] * scale).astype(jnp.int32)
    y1 = jnp.round(r[:, 2] * scale).astype(jnp.int32)
    x2 = jnp.round(r[:, 3] * scale).astype(jnp.int32)
    y2 = jnp.round(r[:, 4] * scale).astype(jnp.int32)
    roi_w = jnp.maximum(x2 - x1 + 1, 1)
    roi_h = jnp.maximum(y2 - y1 + 1, 1)
    bin_h = roi_h.astype(f32) / _P
    bin_w = roi_w.astype(f32) / _P
    ph = jnp.arange(_P, dtype=f32)
    hs = jnp.clip(jnp.floor(ph[None, :] * bin_h[:, None]).astype(jnp.int32)
                  + y1[:, None], 0, H)
    he = jnp.clip(jnp.ceil((ph[None, :] + 1.0) * bin_h[:, None]).astype(jnp.int32)
                  + y1[:, None], 0, H)
    ws = jnp.clip(jnp.floor(ph[None, :] * bin_w[:, None]).astype(jnp.int32)
                  + x1[:, None], 0, W)
    we = jnp.clip(jnp.ceil((ph[None, :] + 1.0) * bin_w[:, None]).astype(jnp.int32)
                  + x1[:, None], 0, W)
    ys = jnp.minimum(y1, H - _K)
    xs = jnp.minimum((jnp.minimum(x1, W - _K) // 8) * 8, W - _KW)
    rowbase = b * H + ys
    cs = ws - xs[:, None]
    ce = we - xs[:, None]
    rs = hs - ys[:, None]
    re = he - ys[:, None]
    c0 = jnp.minimum(cs, _KW - 1)
    c1 = jnp.minimum(cs + 1, _KW - 1)
    c2 = jnp.minimum(cs + 2, _KW - 1)
    wspan = ce - cs
    wq = jnp.stack([c0, c1, c2, wspan], axis=2).reshape(N, 4 * _P)
    r0 = jnp.minimum(rs, _K - 3)
    hp3 = jnp.stack([r0, rs - r0, re - r0], axis=2).reshape(N, 3 * _P)
    meta = jnp.concatenate(
        [rowbase[:, None], xs[:, None],
         jnp.zeros((N, 14), jnp.int32),
         wq, hp3], axis=1)

    feat_t = jnp.transpose(feature, (0, 2, 3, 1)).reshape(B * H, W, C)

    grid_spec = pltpu.PrefetchScalarGridSpec(
        num_scalar_prefetch=1,
        grid=(N // _G,),
        in_specs=[pl.BlockSpec(memory_space=pl.ANY)],
        out_specs=pl.BlockSpec((_G, C, _P * _P), lambda i, meta: (i, 0, 0)),
        scratch_shapes=[
            pltpu.VMEM((2 * _G, _K, _KW, C), f32),
            pltpu.VMEM((_G, _K, _P, C), f32),
            pltpu.VMEM((_G, _P * _P, C), f32),
            pltpu.SemaphoreType.DMA((2 * _G,)),
        ],
    )
    out = pl.pallas_call(
        _roi_kernel,
        grid_spec=grid_spec,
        out_shape=jax.ShapeDtypeStruct((N, C, _P * _P), f32),
    )(meta, feat_t)
    return out.reshape(N, C, _P, _P)


# speedup vs baseline: 3.8129x; 1.1646x over previous
import jax
import jax.numpy as jnp
from jax.experimental import pallas as pl
from jax.experimental.pallas import tpu as pltpu

_P = 7
_K = 14
_KW = 24
_G = 16


def _roi_kernel(meta_ref, feat_hbm, out_ref, crop, wp, res, sems):
    i = pl.program_id(0)
    n = pl.num_programs(0)
    slot = jax.lax.rem(i, 2)
    nslot = jax.lax.rem(i + 1, 2)

    def copy_for(step, s, j):
        roi = _G * step + j
        rb = meta_ref[roi, 0]
        xs = pl.multiple_of(meta_ref[roi, 1], 8)
        return pltpu.make_async_copy(
            feat_hbm.at[pl.ds(rb, _K), pl.ds(xs, _KW), :],
            crop.at[_G * s + j], sems.at[_G * s + j])

    @pl.when(i == 0)
    def _():
        for j in range(_G):
            copy_for(0, 0, j).start()

    @pl.when(i + 1 < n)
    def _():
        for j in range(_G):
            copy_for(i + 1, nslot, j).start()

    for j in range(_G):
        copy_for(i, slot, j).wait()

    c = crop.shape[-1]

    for j in range(_G):
        roi = _G * i + j
        bi = _G * slot + j
        neg1 = jnp.full((_K, 1, c), -jnp.inf, jnp.float32)
        for q in range(_P):
            c0 = meta_ref[roi, 16 + 4 * q]
            c1 = meta_ref[roi, 17 + 4 * q]
            c2 = meta_ref[roi, 18 + 4 * q]
            span = meta_ref[roi, 19 + 4 * q]
            a0 = crop[bi, :, pl.ds(c0, 1), :]
            a1 = crop[bi, :, pl.ds(c1, 1), :]
            a2 = crop[bi, :, pl.ds(c2, 1), :]
            m = a0
            m = jnp.where(span >= 2, jnp.maximum(m, a1), m)
            m = jnp.where(span >= 3, jnp.maximum(m, a2), m)
            m = jnp.where(span >= 1, m, neg1)
            wp[j, :, pl.ds(q, 1), :] = m

        iota_h = jax.lax.broadcasted_iota(jnp.int32, (3, 1, 1), 0)
        for p in range(_P):
            r0 = meta_ref[roi, 44 + 3 * p]
            lo = meta_ref[roi, 45 + 3 * p]
            hi = meta_ref[roi, 46 + 3 * p]
            b = wp[j, pl.ds(r0, 3), :, :]
            b = jnp.where((iota_h >= lo) & (iota_h < hi), b, -jnp.inf)
            m = jnp.max(b, axis=0)
            res[j, pl.ds(_P * p, _P), :] = m

    for j in range(_G):
        v = res[j, :, :]
        v = jnp.where(v == -jnp.inf, jnp.float32(0.0), v)
        out_ref[j] = v.T


def kernel(rois, feature, stride):
    B, C, H, W = feature.shape
    N = rois.shape[0]
    f32 = jnp.float32
    scale = (1.0 / stride)

    r = rois[:, :5].astype(f32)
    b = r[:, 0].astype(jnp.int32)
    x1 = jnp.round(r[:, 1] * scale).astype(jnp.int32)
    y1 = jnp.round(r[:, 2] * scale).astype(jnp.int32)
    x2 = jnp.round(r[:, 3] * scale).astype(jnp.int32)
    y2 = jnp.round(r[:, 4] * scale).astype(jnp.int32)
    roi_w = jnp.maximum(x2 - x1 + 1, 1)
    roi_h = jnp.maximum(y2 - y1 + 1, 1)
    bin_h = roi_h.astype(f32) / _P
    bin_w = roi_w.astype(f32) / _P
    ph = jnp.arange(_P, dtype=f32)
    hs = jnp.clip(jnp.floor(ph[None, :] * bin_h[:, None]).astype(jnp.int32)
                  + y1[:, None], 0, H)
    he = jnp.clip(jnp.ceil((ph[None, :] + 1.0) * bin_h[:, None]).astype(jnp.int32)
                  + y1[:, None], 0, H)
    ws = jnp.clip(jnp.floor(ph[None, :] * bin_w[:, None]).astype(jnp.int32)
                  + x1[:, None], 0, W)
    we = jnp.clip(jnp.ceil((ph[None, :] + 1.0) * bin_w[:, None]).astype(jnp.int32)
                  + x1[:, None], 0, W)
    ys = jnp.minimum(y1, H - _K)
    xs = jnp.minimum((jnp.minimum(x1, W - _K) // 8) * 8, W - _KW)
    rowbase = b * H + ys
    cs = ws - xs[:, None]
    ce = we - xs[:, None]
    rs = hs - ys[:, None]
    re = he - ys[:, None]
    c0 = jnp.minimum(cs, _KW - 1)
    c1 = jnp.minimum(cs + 1, _KW - 1)
    c2 = jnp.minimum(cs + 2, _KW - 1)
    wspan = ce - cs
    wq = jnp.stack([c0, c1, c2, wspan], axis=2).reshape(N, 4 * _P)
    r0 = jnp.minimum(rs, _K - 3)
    hp3 = jnp.stack([r0, rs - r0, re - r0], axis=2).reshape(N, 3 * _P)
    meta = jnp.concatenate(
        [rowbase[:, None], xs[:, None],
         jnp.zeros((N, 14), jnp.int32),
         wq, hp3], axis=1)

    feat_t = jnp.transpose(feature, (0, 2, 3, 1)).reshape(B * H, W, C)

    grid_spec = pltpu.PrefetchScalarGridSpec(
        num_scalar_prefetch=1,
        grid=(N // _G,),
        in_specs=[pl.BlockSpec(memory_space=pl.ANY)],
        out_specs=pl.BlockSpec((_G, C, _P * _P), lambda i, meta: (i, 0, 0)),
        scratch_shapes=[
            pltpu.VMEM((2 * _G, _K, _KW, C), f32),
            pltpu.VMEM((_G, _K, _P, C), f32),
            pltpu.VMEM((_G, _P * _P, C), f32),
            pltpu.SemaphoreType.DMA((2 * _G,)),
        ],
    )
    out = pl.pallas_call(
        _roi_kernel,
        grid_spec=grid_spec,
        out_shape=jax.ShapeDtypeStruct((N, C, _P * _P), f32),
    )(meta, feat_t)
    return out.reshape(N, C, _P, _P)


# speedup vs baseline: 3.8816x; 1.0180x over previous
import jax
import jax.numpy as jnp
from jax.experimental import pallas as pl
from jax.experimental.pallas import tpu as pltpu

_P = 7
_K = 14
_KW = 24
_G = 32


def _roi_kernel(meta_ref, feat_hbm, out_ref, crop, wp, res, sems):
    i = pl.program_id(0)
    n = pl.num_programs(0)
    slot = jax.lax.rem(i, 2)
    nslot = jax.lax.rem(i + 1, 2)

    def copy_for(step, s, j):
        roi = _G * step + j
        rb = meta_ref[roi, 0]
        xs = pl.multiple_of(meta_ref[roi, 1], 8)
        return pltpu.make_async_copy(
            feat_hbm.at[pl.ds(rb, _K), pl.ds(xs, _KW), :],
            crop.at[_G * s + j], sems.at[_G * s + j])

    @pl.when(i == 0)
    def _():
        for j in range(_G):
            copy_for(0, 0, j).start()

    @pl.when(i + 1 < n)
    def _():
        for j in range(_G):
            copy_for(i + 1, nslot, j).start()

    for j in range(_G):
        copy_for(i, slot, j).wait()

    c = crop.shape[-1]

    for j in range(_G):
        roi = _G * i + j
        bi = _G * slot + j
        neg1 = jnp.full((_K, 1, c), -jnp.inf, jnp.float32)
        for q in range(_P):
            c0 = meta_ref[roi, 16 + 4 * q]
            c1 = meta_ref[roi, 17 + 4 * q]
            c2 = meta_ref[roi, 18 + 4 * q]
            span = meta_ref[roi, 19 + 4 * q]
            a0 = crop[bi, :, pl.ds(c0, 1), :]
            a1 = crop[bi, :, pl.ds(c1, 1), :]
            a2 = crop[bi, :, pl.ds(c2, 1), :]
            m = a0
            m = jnp.where(span >= 2, jnp.maximum(m, a1), m)
            m = jnp.where(span >= 3, jnp.maximum(m, a2), m)
            m = jnp.where(span >= 1, m, neg1)
            wp[j, :, pl.ds(q, 1), :] = m

        iota_h = jax.lax.broadcasted_iota(jnp.int32, (3, 1, 1), 0)
        for p in range(_P):
            r0 = meta_ref[roi, 44 + 3 * p]
            lo = meta_ref[roi, 45 + 3 * p]
            hi = meta_ref[roi, 46 + 3 * p]
            b = wp[j, pl.ds(r0, 3), :, :]
            b = jnp.where((iota_h >= lo) & (iota_h < hi), b, -jnp.inf)
            m = jnp.max(b, axis=0)
            res[j, pl.ds(_P * p, _P), :] = m

    for j in range(_G):
        v = res[j, :, :]
        v = jnp.where(v == -jnp.inf, jnp.float32(0.0), v)
        out_ref[j] = v.T


def kernel(rois, feature, stride):
    B, C, H, W = feature.shape
    N = rois.shape[0]
    f32 = jnp.float32
    scale = (1.0 / stride)

    r = rois[:, :5].astype(f32)
    b = r[:, 0].astype(jnp.int32)
    x1 = jnp.round(r[:, 1] * scale).astype(jnp.int32)
    y1 = jnp.round(r[:, 2] * scale).astype(jnp.int32)
    x2 = jnp.round(r[:, 3] * scale).astype(jnp.int32)
    y2 = jnp.round(r[:, 4] * scale).astype(jnp.int32)
    roi_w = jnp.maximum(x2 - x1 + 1, 1)
    roi_h = jnp.maximum(y2 - y1 + 1, 1)
    bin_h = roi_h.astype(f32) / _P
    bin_w = roi_w.astype(f32) / _P
    ph = jnp.arange(_P, dtype=f32)
    hs = jnp.clip(jnp.floor(ph[None, :] * bin_h[:, None]).astype(jnp.int32)
                  + y1[:, None], 0, H)
    he = jnp.clip(jnp.ceil((ph[None, :] + 1.0) * bin_h[:, None]).astype(jnp.int32)
                  + y1[:, None], 0, H)
    ws = jnp.clip(jnp.floor(ph[None, :] * bin_w[:, None]).astype(jnp.int32)
                  + x1[:, None], 0, W)
    we = jnp.clip(jnp.ceil((ph[None, :] + 1.0) * bin_w[:, None]).astype(jnp.int32)
                  + x1[:, None], 0, W)
    ys = jnp.minimum(y1, H - _K)
    xs = jnp.minimum((jnp.minimum(x1, W - _K) // 8) * 8, W - _KW)
    rowbase = b * H + ys
    cs = ws - xs[:, None]
    ce = we - xs[:, None]
    rs = hs - ys[:, None]
    re = he - ys[:, None]
    c0 = jnp.minimum(cs, _KW - 1)
    c1 = jnp.minimum(cs + 1, _KW - 1)
    c2 = jnp.minimum(cs + 2, _KW - 1)
    wspan = ce - cs
    wq = jnp.stack([c0, c1, c2, wspan], axis=2).reshape(N, 4 * _P)
    r0 = jnp.minimum(rs, _K - 3)
    hp3 = jnp.stack([r0, rs - r0, re - r0], axis=2).reshape(N, 3 * _P)
    meta = jnp.concatenate(
        [rowbase[:, None], xs[:, None],
         jnp.zeros((N, 14), jnp.int32),
         wq, hp3], axis=1)

    feat_t = jnp.transpose(feature, (0, 2, 3, 1)).reshape(B * H, W, C)

    grid_spec = pltpu.PrefetchScalarGridSpec(
        num_scalar_prefetch=1,
        grid=(N // _G,),
        in_specs=[pl.BlockSpec(memory_space=pl.ANY)],
        out_specs=pl.BlockSpec((_G, C, _P * _P), lambda i, meta: (i, 0, 0)),
        scratch_shapes=[
            pltpu.VMEM((2 * _G, _K, _KW, C), f32),
            pltpu.VMEM((_G, _K, _P, C), f32),
            pltpu.VMEM((_G, _P * _P, C), f32),
            pltpu.SemaphoreType.DMA((2 * _G,)),
        ],
    )
    out = pl.pallas_call(
        _roi_kernel,
        grid_spec=grid_spec,
        out_shape=jax.ShapeDtypeStruct((N, C, _P * _P), f32),
    )(meta, feat_t)
    return out.reshape(N, C, _P, _P)
